# baseline (speedup 1.0000x reference)
import functools

import jax
import jax.numpy as jnp
from jax import lax
from jax.experimental import pallas as pl
from jax.experimental.pallas import tpu as pltpu
from jax.experimental.pallas import tpu_sc as plsc

N = 10000
E = 160000
NF = 8
NB = 4
NM = 128
NE = 4

NPAD = 10240
EPT = 5120
EPAD = 32 * EPT
ZR = NPAD // 16
NBUF = 4


def _sub(d):
    return 64 if d == 128 else 128


@functools.lru_cache(maxsize=None)
def _make_agg(d, rows_in):
    mesh = plsc.VectorSubcoreMesh(core_axis_name="c", subcore_axis_name="s")
    SUB = _sub(d)
    NS = EPT // SUB

    @functools.partial(
        pl.kernel,
        out_type=jax.ShapeDtypeStruct((2, NPAD, d), jnp.float32),
        mesh=mesh,
        scratch_types=[
            pltpu.VMEM((NS + NBUF, SUB), jnp.int32),
            pltpu.VMEM((NS, SUB), jnp.int32),
            [pltpu.VMEM((SUB, d), jnp.float32) for _ in range(NBUF)],
            pltpu.VMEM_SHARED((NPAD, d), jnp.float32),
            [pltpu.SemaphoreType.DMA for _ in range(NBUF)],
            pltpu.SemaphoreType.DMA,
        ],
        compiler_params=pltpu.CompilerParams(use_tc_tiling_on_sc=False),
    )
    def agg(table, srcw, dstw, zrows, out, idx_s, idx_d, rows, acc, gsem,
            ssem):
        c = lax.axis_index("c")
        s = lax.axis_index("s")
        pltpu.sync_copy(zrows, acc.at[pl.ds(s * ZR, ZR)])
        pltpu.sync_copy(srcw.at[c, s], idx_s.at[pl.ds(0, NS)])
        pltpu.sync_copy(dstw.at[c, s], idx_d)
        zi = jnp.zeros((16,), jnp.int32)
        for jj in range(NBUF):
            for kk in range(SUB // 16):
                idx_s[NS + jj, pl.ds(kk * 16, 16)] = zi
        plsc.subcore_barrier()

        for x in range(NBUF):
            pltpu.async_copy(table.at[idx_s.at[x]], rows[x], gsem[x])

        def body(jj, carry):
            j = jj * NBUF
            for x in range(NBUF):
                pltpu.make_async_copy(
                    table.at[idx_s.at[0]], rows[x], gsem[x]).wait()
                pltpu.async_copy(rows[x], acc.at[idx_d.at[j + x]], ssem,
                                 add=True)
            for x in range(NBUF):
                pltpu.make_async_copy(rows[x], acc.at[idx_d.at[0]],
                                      ssem).wait()
                pltpu.async_copy(table.at[idx_s.at[j + NBUF + x]], rows[x],
                                 gsem[x])
            return carry

        lax.fori_loop(0, NS // NBUF, body, 0)
        for x in range(NBUF):
            pltpu.make_async_copy(table.at[idx_s.at[0]], rows[x],
                                  gsem[x]).wait()
        plsc.subcore_barrier()
        pltpu.sync_copy(acc.at[pl.ds(s * ZR, ZR)], out.at[c, pl.ds(s * ZR, ZR)])

    return agg


def _tc_stage(inputs, body_fn, out_widths, rows=NPAD, bm=1024):
    grid = (rows // bm,)
    in_specs = []
    for a in inputs:
        if a.shape[0] == rows:
            in_specs.append(pl.BlockSpec((bm, a.shape[1]), lambda i: (i, 0)))
        else:
            in_specs.append(pl.BlockSpec(a.shape, lambda i: (0, 0)))
    out_shape = tuple(jax.ShapeDtypeStruct((rows, w), jnp.float32)
                      for w in out_widths)
    out_specs = tuple(pl.BlockSpec((bm, w), lambda i: (i, 0))
                      for w in out_widths)

    def kern(*refs):
        ins = refs[:len(inputs)]
        outs = refs[len(inputs):]
        vals = body_fn(*[r[...] for r in ins])
        if not isinstance(vals, tuple):
            vals = (vals,)
        for o, v in zip(outs, vals):
            o[...] = v

    res = pl.pallas_call(
        kern, grid=grid, in_specs=in_specs, out_specs=out_specs,
        out_shape=out_shape)(*inputs)
    return res


def _dot(x, w):
    return jnp.dot(x, w, preferred_element_type=jnp.float32)


def kernel(x_field, mesh_x, boundary, edge_attr, edge_index, params):
    p = params
    f32 = jnp.float32

    def padN(a):
        return jnp.pad(a, ((0, NPAD - N), (0, 0)))

    xf = padN(x_field)
    mx = padN(mesh_x)
    bd = padN(boundary)
    ea = jnp.pad(edge_attr, ((0, EPAD - E), (0, 0)))
    srcp = jnp.pad(edge_index[0], (0, EPAD - E))
    dstp = jnp.pad(edge_index[1], (0, EPAD - E), constant_values=NPAD - 1)
    eidxp = jnp.arange(EPAD, dtype=jnp.int32)
    zeros = {d: jnp.zeros((ZR, d), f32) for d in (16, 32, 64, 128)}

    def agg(table, idx=None):
        d = table.shape[1]
        sub = _sub(d)
        ns = EPT // sub
        i4 = (srcp if idx is None else idx).reshape(2, 16, ns, sub)
        d4 = dstp.reshape(2, 16, ns, sub)
        return _make_agg(d, table.shape[0])(table, i4, d4, zeros[d])

    def b2(name):
        return p[name].reshape(1, -1)

    def bpad(name, d):
        b = p[name]
        return jnp.pad(b, (0, d - b.shape[0])).reshape(1, -1)

    def wpad(name, d):
        w = p[name]
        return jnp.pad(w, ((0, 0), (0, d - w.shape[1])))

    r = jax.nn.relu

    w_mesh_n = p["W_mesh"][:NM]
    w_mesh_e = p["W_mesh"][NM:]
    (ym,) = _tc_stage([mx, w_mesh_n], lambda x, w: _dot(x, w), (NM,))
    (t_edges,) = _tc_stage([ea, w_mesh_e], lambda x, w: _dot(x, w), (NM,),
                           rows=EPAD, bm=2048)
    am = agg(ym)
    at = agg(t_edges, eidxp)

    (m, yu1) = _tc_stage(
        [am[0], am[1], at[0], at[1], b2("b_mesh"), p["W_u1"]],
        lambda a0, a1, a2, a3, b, w: (
            lambda mm: (mm, _dot(mm, w)))(r(a0 + a1 + a2 + a3 + b)),
        (NM, NM))

    a = agg(yu1)
    (u1, yu2) = _tc_stage(
        [a[0], a[1], b2("b_u1"), m, p["W_u2"]],
        lambda a0, a1, b, res, w: (
            lambda u: (u, _dot(u, w)))(r(a0 + a1 + b) + res),
        (NM, NM))
    a = agg(yu2)
    (u2, yu3) = _tc_stage(
        [a[0], a[1], b2("b_u2"), u1, p["W_u3"]],
        lambda a0, a1, b, res, w: (
            lambda u: (u, _dot(u, w)))(r(a0 + a1 + b) + res),
        (NM, NM))
    a = agg(yu3)
    wd10 = p["W_d10"]
    (yd10,) = _tc_stage(
        [a[0], a[1], b2("b_u3"), u2, xf, bd, wd10[:NF], wd10[NF:NF + NB],
         wd10[NF + NB:]],
        lambda a0, a1, b, res, x, bdv, w1, w2, w3: (
            lambda u: _dot(x, w1) + _dot(bdv, w2) + _dot(u, w3))(
                r(a0 + a1 + b) + res),
        (64,))

    a = agg(yd10)
    (d0, yd11) = _tc_stage(
        [a[0], a[1], b2("b_d10"), p["W_d11"]],
        lambda a0, a1, b, w: (lambda x: (x, _dot(x, w)))(r(a0 + a1 + b)),
        (64, 64))
    a = agg(yd11)
    (yd12,) = _tc_stage(
        [a[0], a[1], b2("b_d11"), p["W_d12"]],
        lambda a0, a1, b, w: _dot(r(a0 + a1 + b), w),
        (64,))
    a = agg(yd12)
    (d2,) = _tc_stage(
        [a[0], a[1], b2("b_d12"), d0],
        lambda a0, a1, b, res: r(a0 + a1 + b) + res,
        (64,))

    a = agg(d2)
    (e0, ye1) = _tc_stage(
        [a[0], a[1], p["W_d20"], b2("b_d20"), p["W_d21"]],
        lambda a0, a1, w0, b, w: (
            lambda x: (x, _dot(x, w)))(r(_dot(a0 + a1, w0) + b)),
        (NM, NM))
    a = agg(ye1)
    (ye2,) = _tc_stage(
        [a[0], a[1], b2("b_d21"), p["W_d22"]],
        lambda a0, a1, b, w: _dot(r(a0 + a1 + b), w),
        (NM,))
    a = agg(ye2)
    (yf0,) = _tc_stage(
        [a[0], a[1], b2("b_d22"), e0, p["W_d30"]],
        lambda a0, a1, b, res, w: _dot(r(a0 + a1 + b) + res, w),
        (NM,))

    a = agg(yf0)
    (yf1,) = _tc_stage(
        [a[0], a[1], b2("b_d30"), p["W_d31"]],
        lambda a0, a1, b, w: _dot(r(a0 + a1 + b), w),
        (64,))
    a = agg(yf1)
    (yf2,) = _tc_stage(
        [a[0], a[1], b2("b_d31"), p["W_d32"]],
        lambda a0, a1, b, w: _dot(r(a0 + a1 + b), w),
        (32,))
    a = agg(yf2)
    (yfd,) = _tc_stage(
        [a[0], a[1], b2("b_d32"), wpad("W_fdot", 16)],
        lambda a0, a1, b, w: _dot(r(a0 + a1 + b), w),
        (16,))
    a = agg(yfd)
    (fdot,) = _tc_stage(
        [a[0], a[1], bpad("b_fdot", 16)],
        lambda a0, a1, b: a0 + a1 + b,
        (16,))

    a = agg(fdot)
    (i0, yi1) = _tc_stage(
        [a[0], a[1], jnp.pad(p["W_i10"], ((0, 8), (0, 0))), b2("b_i10"),
         p["W_i11"]],
        lambda a0, a1, w0, b, w: (
            lambda x: (x, _dot(x, w)))(r(_dot(a0 + a1, w0) + b)),
        (64, 64))
    a = agg(yi1)
    (yi2,) = _tc_stage(
        [a[0], a[1], b2("b_i11"), p["W_i12"]],
        lambda a0, a1, b, w: _dot(r(a0 + a1 + b), w),
        (64,))
    a = agg(yi2)
    (yio,) = _tc_stage(
        [a[0], a[1], b2("b_i12"), i0, wpad("W_iout", 16)],
        lambda a0, a1, b, res, w: _dot(r(a0 + a1 + b) + res, w),
        (16,))
    a = agg(yio)
    (out,) = _tc_stage(
        [a[0], a[1], bpad("b_iout", 16), xf],
        lambda a0, a1, b, x: x + (a0 + a1 + b)[:, :NF],
        (NF,))

    return out[:N]


# speedup vs baseline: 3.2890x; 3.2890x over previous
import functools

import jax
import jax.numpy as jnp
from jax import lax
from jax.experimental import pallas as pl
from jax.experimental.pallas import tpu as pltpu
from jax.experimental.pallas import tpu_sc as plsc

N = 10000
E = 160000
NF = 8
NB = 4
NM = 128
NE = 4

NPAD = 10240
SUB = 128
EPT = 5120
EPAD = 32 * EPT
EPT2 = 2 * EPT
NS = EPT // SUB
NS2 = EPT2 // SUB
ZR = NPAD // 16
ZB = 64


def _zero_acc(zblk, acc, s, dh):
    zf = jnp.zeros((16,), jnp.float32)

    def zb(i, carry):
        for kk in range(dh // 16):
            zblk[i, pl.ds(kk * 16, 16)] = zf
        return carry

    lax.fori_loop(0, ZB, zb, 0)
    for t in range(ZR // ZB):
        pltpu.sync_copy(zblk, acc.at[pl.ds(s * ZR + t * ZB, ZB)])


@functools.lru_cache(maxsize=None)
def _make_agg_split(dh, with_edge_tab=False):
    mesh = plsc.VectorSubcoreMesh(core_axis_name="c", subcore_axis_name="s")
    scratch = [
        pltpu.VMEM((NS2, SUB), jnp.int32),
        pltpu.VMEM((NS2, SUB), jnp.int32),
        pltpu.VMEM((SUB, dh), jnp.float32),
        pltpu.VMEM((ZB, dh), jnp.float32),
        pltpu.VMEM_SHARED((NPAD, dh), jnp.float32),
        pltpu.VMEM_SHARED((NPAD, dh), jnp.float32),
    ]

    def body(tbl2, srcw, dstw, *rest):
        if with_edge_tab:
            t2 = rest[0]
            rest = rest[1:]
        out, idx_s, idx_d, rows, zblk, tbl, acc = rest
        c = lax.axis_index("c")
        s = lax.axis_index("s")
        pltpu.sync_copy(srcw.at[s], idx_s)
        pltpu.sync_copy(dstw.at[s], idx_d)
        pltpu.sync_copy(tbl2.at[c, pl.ds(s * ZR, ZR)], tbl.at[pl.ds(s * ZR, ZR)])
        _zero_acc(zblk, acc, s, dh)
        plsc.subcore_barrier()

        def step(j, carry):
            pltpu.sync_copy(tbl.at[idx_s.at[j]], rows)
            pltpu.sync_copy(rows, acc.at[idx_d.at[j]], add=True)
            return carry

        lax.fori_loop(0, NS2, step, 0)
        if with_edge_tab:
            base = s * EPT2

            def estep(j, carry):
                pltpu.sync_copy(t2.at[c, pl.ds(base + j * SUB, SUB)], rows)
                pltpu.sync_copy(rows, acc.at[idx_d.at[j]], add=True)
                return carry

            lax.fori_loop(0, NS2, estep, 0)
        plsc.subcore_barrier()
        pltpu.sync_copy(acc.at[pl.ds(s * ZR, ZR)], out.at[c, pl.ds(s * ZR, ZR)])

    return pl.kernel(
        body,
        out_type=jax.ShapeDtypeStruct((2, NPAD, dh), jnp.float32),
        mesh=mesh,
        scratch_types=scratch,
        compiler_params=pltpu.CompilerParams(use_tc_tiling_on_sc=False),
    )


@functools.lru_cache(maxsize=None)
def _make_agg16():
    d = 16
    mesh = plsc.VectorSubcoreMesh(core_axis_name="c", subcore_axis_name="s")
    scratch = [
        pltpu.VMEM((NS, SUB), jnp.int32),
        pltpu.VMEM((NS, SUB), jnp.int32),
        pltpu.VMEM((SUB, d), jnp.float32),
        pltpu.VMEM((ZB, d), jnp.float32),
        pltpu.VMEM_SHARED((NPAD, d), jnp.float32),
        pltpu.VMEM_SHARED((NPAD, d), jnp.float32),
    ]

    def body(table, srcw, dstw, out, idx_s, idx_d, rows, zblk, tbl, acc):
        c = lax.axis_index("c")
        s = lax.axis_index("s")
        pltpu.sync_copy(srcw.at[c, s], idx_s)
        pltpu.sync_copy(dstw.at[c, s], idx_d)
        pltpu.sync_copy(table.at[pl.ds(s * ZR, ZR)], tbl.at[pl.ds(s * ZR, ZR)])
        _zero_acc(zblk, acc, s, d)
        plsc.subcore_barrier()

        def step(j, carry):
            pltpu.sync_copy(tbl.at[idx_s.at[j]], rows)
            pltpu.sync_copy(rows, acc.at[idx_d.at[j]], add=True)
            return carry

        lax.fori_loop(0, NS, step, 0)
        plsc.subcore_barrier()
        pltpu.sync_copy(acc.at[pl.ds(s * ZR, ZR)], out.at[c, pl.ds(s * ZR, ZR)])

    return pl.kernel(
        body,
        out_type=jax.ShapeDtypeStruct((2, NPAD, d), jnp.float32),
        mesh=mesh,
        scratch_types=scratch,
        compiler_params=pltpu.CompilerParams(use_tc_tiling_on_sc=False),
    )


def _tc_stage(inputs, body_fn, out_widths, rows=NPAD, bm=1024):
    grid = (rows // bm,)
    in_specs = []
    for a in inputs:
        if a.shape[0] == rows:
            in_specs.append(pl.BlockSpec((bm, a.shape[1]), lambda i: (i, 0)))
        else:
            in_specs.append(pl.BlockSpec(a.shape, lambda i: (0, 0)))
    out_shape = tuple(jax.ShapeDtypeStruct((rows, w), jnp.float32)
                      for w in out_widths)
    out_specs = tuple(pl.BlockSpec((bm, w), lambda i: (i, 0))
                      for w in out_widths)

    def kern(*refs):
        ins = refs[:len(inputs)]
        outs = refs[len(inputs):]
        vals = body_fn(*[r[...] for r in ins])
        if not isinstance(vals, tuple):
            vals = (vals,)
        for o, v in zip(outs, vals):
            o[...] = v

    return pl.pallas_call(
        kern, grid=grid, in_specs=in_specs, out_specs=out_specs,
        out_shape=out_shape)(*inputs)


def _dot(x, w):
    return jnp.dot(x, w, preferred_element_type=jnp.float32)


def _cat(a, b):
    return jnp.concatenate([a, b], axis=-1)


def kernel(x_field, mesh_x, boundary, edge_attr, edge_index, params):
    p = params
    f32 = jnp.float32

    def padN(a):
        return jnp.pad(a, ((0, NPAD - N), (0, 0)))

    xf = padN(x_field)
    mx = padN(mesh_x)
    bd = padN(boundary)
    ea = jnp.pad(edge_attr, ((0, EPAD - E), (0, 0)))
    srcp = jnp.pad(edge_index[0], (0, EPAD - E))
    dstp = jnp.pad(edge_index[1], (0, EPAD - E), constant_values=NPAD - 1)
    src16 = srcp.reshape(16, NS2, SUB)
    dst16 = dstp.reshape(16, NS2, SUB)
    src32 = srcp.reshape(2, 16, NS, SUB)
    dst32 = dstp.reshape(2, 16, NS, SUB)

    def agg(table2):
        return _make_agg_split(table2.shape[2])(table2, src16, dst16)

    def agg16(table):
        return _make_agg16()(table, src32, dst32)

    def b2(name):
        return p[name].reshape(1, -1)

    def bpad(name, d):
        b = p[name]
        return jnp.pad(b, (0, d - b.shape[0])).reshape(1, -1)

    def wpad(name, d):
        w = p[name]
        return jnp.pad(w, ((0, 0), (0, d - w.shape[1])))

    r = jax.nn.relu

    w_mesh_n = p["W_mesh"][:NM]
    w_mesh_e = p["W_mesh"][NM:]
    (yml, ymr) = _tc_stage(
        [mx, w_mesh_n],
        lambda x, w: (_dot(x, w[:, :64]), _dot(x, w[:, 64:])),
        (64, 64))
    (tl, tr) = _tc_stage(
        [ea, w_mesh_e],
        lambda x, w: (_dot(x, w[:, :64]), _dot(x, w[:, 64:])),
        (64, 64), rows=EPAD, bm=4096)
    ym2 = jnp.stack([yml, ymr])
    t2 = jnp.stack([tl, tr])
    am = _make_agg_split(64, with_edge_tab=True)(ym2, src16, dst16, t2)

    (m, yu1) = _tc_stage(
        [am[0], am[1], b2("b_mesh"), p["W_u1"]],
        lambda a0, a1, b, w: (
            lambda mm: (mm, _dot(mm, w)))(r(_cat(a0, a1) + b)),
        (NM, NM))

    def split2(y):
        return jnp.stack([y[:, :64], y[:, 64:]])

    a = agg(split2(yu1))
    (u1, yu2) = _tc_stage(
        [a[0], a[1], b2("b_u1"), m, p["W_u2"]],
        lambda a0, a1, b, res, w: (
            lambda u: (u, _dot(u, w)))(r(_cat(a0, a1) + b) + res),
        (NM, NM))
    a = agg(split2(yu2))
    (u2, yu3) = _tc_stage(
        [a[0], a[1], b2("b_u2"), u1, p["W_u3"]],
        lambda a0, a1, b, res, w: (
            lambda u: (u, _dot(u, w)))(r(_cat(a0, a1) + b) + res),
        (NM, NM))
    a = agg(split2(yu3))
    wd10 = p["W_d10"]
    (yd10,) = _tc_stage(
        [a[0], a[1], b2("b_u3"), u2, xf, bd, wd10[:NF], wd10[NF:NF + NB],
         wd10[NF + NB:]],
        lambda a0, a1, b, res, x, bdv, w1, w2, w3: (
            lambda u: _dot(x, w1) + _dot(bdv, w2) + _dot(u, w3))(
                r(_cat(a0, a1) + b) + res),
        (64,))

    def split64(y):
        return jnp.stack([y[:, :32], y[:, 32:]])

    a = agg(split64(yd10))
    (d0, yd11) = _tc_stage(
        [a[0], a[1], b2("b_d10"), p["W_d11"]],
        lambda a0, a1, b, w: (
            lambda x: (x, _dot(x, w)))(r(_cat(a0, a1) + b)),
        (64, 64))
    a = agg(split64(yd11))
    (yd12,) = _tc_stage(
        [a[0], a[1], b2("b_d11"), p["W_d12"]],
        lambda a0, a1, b, w: _dot(r(_cat(a0, a1) + b), w),
        (64,))
    a = agg(split64(yd12))
    (d2,) = _tc_stage(
        [a[0], a[1], b2("b_d12"), d0],
        lambda a0, a1, b, res: r(_cat(a0, a1) + b) + res,
        (64,))

    a = agg(split64(d2))
    (e0, ye1) = _tc_stage(
        [a[0], a[1], p["W_d20"], b2("b_d20"), p["W_d21"]],
        lambda a0, a1, w0, b, w: (
            lambda x: (x, _dot(x, w)))(r(_dot(_cat(a0, a1), w0) + b)),
        (NM, NM))
    a = agg(split2(ye1))
    (ye2,) = _tc_stage(
        [a[0], a[1], b2("b_d21"), p["W_d22"]],
        lambda a0, a1, b, w: _dot(r(_cat(a0, a1) + b), w),
        (NM,))
    a = agg(split2(ye2))
    (yf0,) = _tc_stage(
        [a[0], a[1], b2("b_d22"), e0, p["W_d30"]],
        lambda a0, a1, b, res, w: _dot(r(_cat(a0, a1) + b) + res, w),
        (NM,))

    a = agg(split2(yf0))
    (yf1,) = _tc_stage(
        [a[0], a[1], b2("b_d30"), p["W_d31"]],
        lambda a0, a1, b, w: _dot(r(_cat(a0, a1) + b), w),
        (64,))
    a = agg(split64(yf1))
    (yf2,) = _tc_stage(
        [a[0], a[1], b2("b_d31"), p["W_d32"]],
        lambda a0, a1, b, w: _dot(r(_cat(a0, a1) + b), w),
        (32,))
    a = agg(jnp.stack([yf2[:, :16], yf2[:, 16:]]))
    (yfd,) = _tc_stage(
        [a[0], a[1], b2("b_d32"), wpad("W_fdot", 16)],
        lambda a0, a1, b, w: _dot(r(_cat(a0, a1) + b), w),
        (16,))
    a = agg16(yfd)
    (fdot,) = _tc_stage(
        [a[0], a[1], bpad("b_fdot", 16)],
        lambda a0, a1, b: a0 + a1 + b,
        (16,))

    a = agg16(fdot)
    (i0, yi1) = _tc_stage(
        [a[0], a[1], jnp.pad(p["W_i10"], ((0, 8), (0, 0))), b2("b_i10"),
         p["W_i11"]],
        lambda a0, a1, w0, b, w: (
            lambda x: (x, _dot(x, w)))(r(_dot(a0 + a1, w0) + b)),
        (64, 64))
    a = agg(split64(yi1))
    (yi2,) = _tc_stage(
        [a[0], a[1], b2("b_i11"), p["W_i12"]],
        lambda a0, a1, b, w: _dot(r(_cat(a0, a1) + b), w),
        (64,))
    a = agg(split64(yi2))
    (yio,) = _tc_stage(
        [a[0], a[1], b2("b_i12"), i0, wpad("W_iout", 16)],
        lambda a0, a1, b, res, w: _dot(r(_cat(a0, a1) + b) + res, w),
        (16,))
    a = agg16(yio)
    (out,) = _tc_stage(
        [a[0], a[1], bpad("b_iout", 16), xf],
        lambda a0, a1, b, x: x + (a0 + a1 + b)[:, :NF],
        (NF,))

    return out[:N]


# speedup vs baseline: 3.6241x; 1.1019x over previous
import functools

import jax
import jax.numpy as jnp
from jax import lax
from jax.experimental import pallas as pl
from jax.experimental.pallas import tpu as pltpu
from jax.experimental.pallas import tpu_sc as plsc

N = 10000
E = 160000
NF = 8
NB = 4
NM = 128
NE = 4

NPAD = 10240
SUB = 128
EPT = 5120
EPAD = 32 * EPT
EPT2 = 2 * EPT
NS = EPT // SUB
NS2 = EPT2 // SUB
ZR = NPAD // 16
ZB = 64


def _zero_acc(zblk, acc, s, dh):
    zf = jnp.zeros((16,), jnp.float32)

    def zb(i, carry):
        for kk in range(dh // 16):
            zblk[i, pl.ds(kk * 16, 16)] = zf
        return carry

    lax.fori_loop(0, ZB, zb, 0)
    for t in range(ZR // ZB):
        pltpu.sync_copy(zblk, acc.at[pl.ds(s * ZR + t * ZB, ZB)])


@functools.lru_cache(maxsize=None)
def _make_agg_split(dh, with_edge_tab=False):
    mesh = plsc.VectorSubcoreMesh(core_axis_name="c", subcore_axis_name="s")
    scratch = [
        pltpu.VMEM((NS2, SUB), jnp.int32),
        pltpu.VMEM((NS2, SUB), jnp.int32),
        pltpu.VMEM((SUB, dh), jnp.float32),
        pltpu.VMEM((ZB, dh), jnp.float32),
        pltpu.VMEM_SHARED((NPAD, dh), jnp.float32),
        pltpu.VMEM_SHARED((NPAD, dh), jnp.float32),
    ]

    def body(tl, tr, srcw, dstw, *rest):
        if with_edge_tab:
            el, er = rest[0], rest[1]
            rest = rest[2:]
        out, idx_s, idx_d, rows, zblk, tbl, acc = rest
        c = lax.axis_index("c")
        s = lax.axis_index("s")
        pltpu.sync_copy(srcw.at[s], idx_s)
        pltpu.sync_copy(dstw.at[s], idx_d)
        sl = pl.ds(s * ZR, ZR)

        @pl.when(c == 0)
        def _():
            pltpu.sync_copy(tl.at[sl], tbl.at[sl])

        @pl.when(c == 1)
        def _():
            pltpu.sync_copy(tr.at[sl], tbl.at[sl])

        _zero_acc(zblk, acc, s, dh)
        plsc.subcore_barrier()

        def step(j, carry):
            pltpu.sync_copy(tbl.at[idx_s.at[j]], rows)
            pltpu.sync_copy(rows, acc.at[idx_d.at[j]], add=True)
            return carry

        lax.fori_loop(0, NS2, step, 0)
        if with_edge_tab:
            base = s * EPT2

            def estep0(j, carry):
                pltpu.sync_copy(el.at[pl.ds(base + j * SUB, SUB)], rows)
                pltpu.sync_copy(rows, acc.at[idx_d.at[j]], add=True)
                return carry

            def estep1(j, carry):
                pltpu.sync_copy(er.at[pl.ds(base + j * SUB, SUB)], rows)
                pltpu.sync_copy(rows, acc.at[idx_d.at[j]], add=True)
                return carry

            @pl.when(c == 0)
            def _():
                lax.fori_loop(0, NS2, estep0, 0)

            @pl.when(c == 1)
            def _():
                lax.fori_loop(0, NS2, estep1, 0)

        plsc.subcore_barrier()
        pltpu.sync_copy(acc.at[sl], out.at[c, sl])

    return pl.kernel(
        body,
        out_type=jax.ShapeDtypeStruct((2, NPAD, dh), jnp.float32),
        mesh=mesh,
        scratch_types=scratch,
        compiler_params=pltpu.CompilerParams(use_tc_tiling_on_sc=False),
    )


@functools.lru_cache(maxsize=None)
def _make_agg16():
    d = 16
    mesh = plsc.VectorSubcoreMesh(core_axis_name="c", subcore_axis_name="s")
    scratch = [
        pltpu.VMEM((NS, SUB), jnp.int32),
        pltpu.VMEM((NS, SUB), jnp.int32),
        pltpu.VMEM((SUB, d), jnp.float32),
        pltpu.VMEM((ZB, d), jnp.float32),
        pltpu.VMEM_SHARED((NPAD, d), jnp.float32),
        pltpu.VMEM_SHARED((NPAD, d), jnp.float32),
    ]

    def body(table, srcw, dstw, out, idx_s, idx_d, rows, zblk, tbl, acc):
        c = lax.axis_index("c")
        s = lax.axis_index("s")
        pltpu.sync_copy(srcw.at[c, s], idx_s)
        pltpu.sync_copy(dstw.at[c, s], idx_d)
        sl = pl.ds(s * ZR, ZR)
        pltpu.sync_copy(table.at[sl], tbl.at[sl])
        _zero_acc(zblk, acc, s, d)
        plsc.subcore_barrier()

        def step(j, carry):
            pltpu.sync_copy(tbl.at[idx_s.at[j]], rows)
            pltpu.sync_copy(rows, acc.at[idx_d.at[j]], add=True)
            return carry

        lax.fori_loop(0, NS, step, 0)
        plsc.subcore_barrier()
        pltpu.sync_copy(acc.at[sl], out.at[c, sl])

    return pl.kernel(
        body,
        out_type=jax.ShapeDtypeStruct((2, NPAD, d), jnp.float32),
        mesh=mesh,
        scratch_types=scratch,
        compiler_params=pltpu.CompilerParams(use_tc_tiling_on_sc=False),
    )


def _tc_stage(inputs, body_fn, out_widths, rows=NPAD, bm=1024):
    grid = (rows // bm,)
    in_specs = []
    for a in inputs:
        if a.shape[0] == rows:
            in_specs.append(pl.BlockSpec((bm, a.shape[1]), lambda i: (i, 0)))
        else:
            in_specs.append(pl.BlockSpec(a.shape, lambda i: (0, 0)))
    out_shape = tuple(jax.ShapeDtypeStruct((rows, w), jnp.float32)
                      for w in out_widths)
    out_specs = tuple(pl.BlockSpec((bm, w), lambda i: (i, 0))
                      for w in out_widths)

    def kern(*refs):
        ins = refs[:len(inputs)]
        outs = refs[len(inputs):]
        vals = body_fn(*[r[...] for r in ins])
        if not isinstance(vals, tuple):
            vals = (vals,)
        for o, v in zip(outs, vals):
            o[...] = v

    return pl.pallas_call(
        kern, grid=grid, in_specs=in_specs, out_specs=out_specs,
        out_shape=out_shape)(*inputs)


def _dot(x, w):
    return jnp.dot(x, w, preferred_element_type=jnp.float32)


def _cat(a, b):
    return jnp.concatenate([a, b], axis=-1)


def _halves(x):
    h = x.shape[-1] // 2
    return x[:, :h], x[:, h:]


def kernel(x_field, mesh_x, boundary, edge_attr, edge_index, params):
    p = params

    def padN(a):
        return jnp.pad(a, ((0, NPAD - N), (0, 0)))

    xf = padN(x_field)
    mx = padN(mesh_x)
    bd = padN(boundary)
    ea = jnp.pad(edge_attr, ((0, EPAD - E), (0, 0)))
    srcp = jnp.pad(edge_index[0], (0, EPAD - E))
    dstp = jnp.pad(edge_index[1], (0, EPAD - E), constant_values=NPAD - 1)
    src16 = srcp.reshape(16, NS2, SUB)
    dst16 = dstp.reshape(16, NS2, SUB)
    src32 = srcp.reshape(2, 16, NS, SUB)
    dst32 = dstp.reshape(2, 16, NS, SUB)

    def agg(tl, tr):
        return _make_agg_split(tl.shape[1])(tl, tr, src16, dst16)

    def agg16(table):
        return _make_agg16()(table, src32, dst32)

    def b2(name):
        return p[name].reshape(1, -1)

    def bpad(name, d):
        b = p[name]
        return jnp.pad(b, (0, d - b.shape[0])).reshape(1, -1)

    def wpad(name, d):
        w = p[name]
        return jnp.pad(w, ((0, 0), (0, d - w.shape[1])))

    r = jax.nn.relu

    w_mesh_n = p["W_mesh"][:NM]
    w_mesh_e = p["W_mesh"][NM:]
    (yml, ymr) = _tc_stage(
        [mx, w_mesh_n],
        lambda x, w: _halves(_dot(x, w)),
        (64, 64))
    (tel, ter) = _tc_stage(
        [ea, w_mesh_e],
        lambda x, w: _halves(_dot(x, w)),
        (64, 64), rows=EPAD, bm=4096)
    am = _make_agg_split(64, with_edge_tab=True)(yml, ymr, src16, dst16,
                                                 tel, ter)

    (m, y0, y1) = _tc_stage(
        [am[0], am[1], b2("b_mesh"), p["W_u1"]],
        lambda a0, a1, b, w: (
            lambda mm: (mm,) + _halves(_dot(mm, w)))(r(_cat(a0, a1) + b)),
        (NM, 64, 64))

    a = agg(y0, y1)
    (u1, y0, y1) = _tc_stage(
        [a[0], a[1], b2("b_u1"), m, p["W_u2"]],
        lambda a0, a1, b, res, w: (
            lambda u: (u,) + _halves(_dot(u, w)))(r(_cat(a0, a1) + b) + res),
        (NM, 64, 64))
    a = agg(y0, y1)
    (u2, y0, y1) = _tc_stage(
        [a[0], a[1], b2("b_u2"), u1, p["W_u3"]],
        lambda a0, a1, b, res, w: (
            lambda u: (u,) + _halves(_dot(u, w)))(r(_cat(a0, a1) + b) + res),
        (NM, 64, 64))
    a = agg(y0, y1)
    wd10 = p["W_d10"]
    (y0, y1) = _tc_stage(
        [a[0], a[1], b2("b_u3"), u2, xf, bd, wd10[:NF], wd10[NF:NF + NB],
         wd10[NF + NB:]],
        lambda a0, a1, b, res, x, bdv, w1, w2, w3: _halves(
            (lambda u: _dot(x, w1) + _dot(bdv, w2) + _dot(u, w3))(
                r(_cat(a0, a1) + b) + res)),
        (32, 32))

    a = agg(y0, y1)
    (d0, y0, y1) = _tc_stage(
        [a[0], a[1], b2("b_d10"), p["W_d11"]],
        lambda a0, a1, b, w: (
            lambda x: (x,) + _halves(_dot(x, w)))(r(_cat(a0, a1) + b)),
        (64, 32, 32))
    a = agg(y0, y1)
    (y0, y1) = _tc_stage(
        [a[0], a[1], b2("b_d11"), p["W_d12"]],
        lambda a0, a1, b, w: _halves(_dot(r(_cat(a0, a1) + b), w)),
        (32, 32))
    a = agg(y0, y1)
    (y0, y1) = _tc_stage(
        [a[0], a[1], b2("b_d12"), d0],
        lambda a0, a1, b, res: _halves(r(_cat(a0, a1) + b) + res),
        (32, 32))

    a = agg(y0, y1)
    (e0, y0, y1) = _tc_stage(
        [a[0], a[1], p["W_d20"], b2("b_d20"), p["W_d21"]],
        lambda a0, a1, w0, b, w: (
            lambda x: (x,) + _halves(_dot(x, w)))(
                r(_dot(_cat(a0, a1), w0) + b)),
        (NM, 64, 64))
    a = agg(y0, y1)
    (y0, y1) = _tc_stage(
        [a[0], a[1], b2("b_d21"), p["W_d22"]],
        lambda a0, a1, b, w: _halves(_dot(r(_cat(a0, a1) + b), w)),
        (64, 64))
    a = agg(y0, y1)
    (y0, y1) = _tc_stage(
        [a[0], a[1], b2("b_d22"), e0, p["W_d30"]],
        lambda a0, a1, b, res, w: _halves(
            _dot(r(_cat(a0, a1) + b) + res, w)),
        (64, 64))

    a = agg(y0, y1)
    (y0, y1) = _tc_stage(
        [a[0], a[1], b2("b_d30"), p["W_d31"]],
        lambda a0, a1, b, w: _halves(_dot(r(_cat(a0, a1) + b), w)),
        (32, 32))
    a = agg(y0, y1)
    (y0, y1) = _tc_stage(
        [a[0], a[1], b2("b_d31"), p["W_d32"]],
        lambda a0, a1, b, w: _halves(_dot(r(_cat(a0, a1) + b), w)),
        (16, 16))
    a = agg(y0, y1)
    (yfd,) = _tc_stage(
        [a[0], a[1], b2("b_d32"), wpad("W_fdot", 16)],
        lambda a0, a1, b, w: _dot(r(_cat(a0, a1) + b), w),
        (16,))
    a = agg16(yfd)
    (fdot,) = _tc_stage(
        [a[0], a[1], bpad("b_fdot", 16)],
        lambda a0, a1, b: a0 + a1 + b,
        (16,))

    a = agg16(fdot)
    (i0, y0, y1) = _tc_stage(
        [a[0], a[1], jnp.pad(p["W_i10"], ((0, 8), (0, 0))), b2("b_i10"),
         p["W_i11"]],
        lambda a0, a1, w0, b, w: (
            lambda x: (x,) + _halves(_dot(x, w)))(r(_dot(a0 + a1, w0) + b)),
        (64, 32, 32))
    a = agg(y0, y1)
    (y0, y1) = _tc_stage(
        [a[0], a[1], b2("b_i11"), p["W_i12"]],
        lambda a0, a1, b, w: _halves(_dot(r(_cat(a0, a1) + b), w)),
        (32, 32))
    a = agg(y0, y1)
    (yio,) = _tc_stage(
        [a[0], a[1], b2("b_i12"), i0, wpad("W_iout", 16)],
        lambda a0, a1, b, res, w: _dot(r(_cat(a0, a1) + b) + res, w),
        (16,))
    a = agg16(yio)
    (out,) = _tc_stage(
        [a[0], a[1], bpad("b_iout", 16), xf],
        lambda a0, a1, b, x: x + (a0 + a1 + b)[:, :NF],
        (NF,))

    return out[:N]


# speedup vs baseline: 4.2177x; 1.1638x over previous
import functools

import jax
import jax.numpy as jnp
from jax import lax
from jax.experimental import pallas as pl
from jax.experimental.pallas import tpu as pltpu
from jax.experimental.pallas import tpu_sc as plsc

N = 10000
E = 160000
NF = 8
NB = 4
NM = 128
NE = 4

NPAD = 10240
SUB = 128
EPT = 5120
EPAD = 32 * EPT
EPT2 = 2 * EPT
NS = EPT // SUB
NS2 = EPT2 // SUB
ZR = NPAD // 16
ZB = 64


def _zero_acc(zblk, acc, s, dh):
    zf = jnp.zeros((16,), jnp.float32)

    def zb(i, carry):
        for kk in range(dh // 16):
            zblk[i, pl.ds(kk * 16, 16)] = zf
        return carry

    lax.fori_loop(0, ZB, zb, 0)
    for t in range(ZR // ZB):
        pltpu.sync_copy(zblk, acc.at[pl.ds(s * ZR + t * ZB, ZB)])


@functools.lru_cache(maxsize=None)
def _make_agg_split(dh):
    mesh = plsc.VectorSubcoreMesh(core_axis_name="c", subcore_axis_name="s")
    scratch = [
        pltpu.VMEM((NS2, SUB), jnp.int32),
        pltpu.VMEM((NS2, SUB), jnp.int32),
        pltpu.VMEM((SUB, dh), jnp.float32),
        pltpu.VMEM((ZB, dh), jnp.float32),
        pltpu.VMEM_SHARED((NPAD, dh), jnp.float32),
        pltpu.VMEM_SHARED((NPAD, dh), jnp.float32),
    ]

    def body(tl, tr, srcw, dstw, *rest):
        out, idx_s, idx_d, rows, zblk, tbl, acc = rest
        c = lax.axis_index("c")
        s = lax.axis_index("s")
        pltpu.sync_copy(srcw.at[s], idx_s)
        pltpu.sync_copy(dstw.at[s], idx_d)
        sl = pl.ds(s * ZR, ZR)

        @pl.when(c == 0)
        def _():
            pltpu.sync_copy(tl.at[sl], tbl.at[sl])

        @pl.when(c == 1)
        def _():
            pltpu.sync_copy(tr.at[sl], tbl.at[sl])

        _zero_acc(zblk, acc, s, dh)
        plsc.subcore_barrier()

        def step(j, carry):
            pltpu.sync_copy(tbl.at[idx_s.at[j]], rows)
            pltpu.sync_copy(rows, acc.at[idx_d.at[j]], add=True)
            return carry

        lax.fori_loop(0, NS2, step, 0)
        plsc.subcore_barrier()
        pltpu.sync_copy(acc.at[sl], out.at[c, sl])

    return pl.kernel(
        body,
        out_type=jax.ShapeDtypeStruct((2, NPAD, dh), jnp.float32),
        mesh=mesh,
        scratch_types=scratch,
        compiler_params=pltpu.CompilerParams(use_tc_tiling_on_sc=False),
    )


@functools.lru_cache(maxsize=None)
def _make_agg_lin16():
    d = 16
    mesh = plsc.VectorSubcoreMesh(core_axis_name="c", subcore_axis_name="s")
    scratch = [
        pltpu.VMEM((NS, SUB), jnp.int32),
        pltpu.VMEM((SUB, d), jnp.float32),
        pltpu.VMEM((ZB, d), jnp.float32),
        pltpu.VMEM_SHARED((NPAD, d), jnp.float32),
    ]

    def body(etab, dstw, out, idx_d, rows, zblk, acc):
        c = lax.axis_index("c")
        s = lax.axis_index("s")
        pltpu.sync_copy(dstw.at[c, s], idx_d)
        _zero_acc(zblk, acc, s, d)
        base = (c * 16 + s) * EPT
        plsc.subcore_barrier()

        def step(j, carry):
            pltpu.sync_copy(etab.at[pl.ds(base + j * SUB, SUB)], rows)
            pltpu.sync_copy(rows, acc.at[idx_d.at[j]], add=True)
            return carry

        lax.fori_loop(0, NS, step, 0)
        plsc.subcore_barrier()
        sl = pl.ds(s * ZR, ZR)
        pltpu.sync_copy(acc.at[sl], out.at[c, sl])

    return pl.kernel(
        body,
        out_type=jax.ShapeDtypeStruct((2, NPAD, d), jnp.float32),
        mesh=mesh,
        scratch_types=scratch,
        compiler_params=pltpu.CompilerParams(use_tc_tiling_on_sc=False),
    )


@functools.lru_cache(maxsize=None)
def _make_agg16():
    d = 16
    mesh = plsc.VectorSubcoreMesh(core_axis_name="c", subcore_axis_name="s")
    scratch = [
        pltpu.VMEM((NS, SUB), jnp.int32),
        pltpu.VMEM((NS, SUB), jnp.int32),
        pltpu.VMEM((SUB, d), jnp.float32),
        pltpu.VMEM((ZB, d), jnp.float32),
        pltpu.VMEM_SHARED((NPAD, d), jnp.float32),
        pltpu.VMEM_SHARED((NPAD, d), jnp.float32),
    ]

    def body(table, srcw, dstw, out, idx_s, idx_d, rows, zblk, tbl, acc):
        c = lax.axis_index("c")
        s = lax.axis_index("s")
        pltpu.sync_copy(srcw.at[c, s], idx_s)
        pltpu.sync_copy(dstw.at[c, s], idx_d)
        sl = pl.ds(s * ZR, ZR)
        pltpu.sync_copy(table.at[sl], tbl.at[sl])
        _zero_acc(zblk, acc, s, d)
        plsc.subcore_barrier()

        def step(j, carry):
            pltpu.sync_copy(tbl.at[idx_s.at[j]], rows)
            pltpu.sync_copy(rows, acc.at[idx_d.at[j]], add=True)
            return carry

        lax.fori_loop(0, NS, step, 0)
        plsc.subcore_barrier()
        pltpu.sync_copy(acc.at[sl], out.at[c, sl])

    return pl.kernel(
        body,
        out_type=jax.ShapeDtypeStruct((2, NPAD, d), jnp.float32),
        mesh=mesh,
        scratch_types=scratch,
        compiler_params=pltpu.CompilerParams(use_tc_tiling_on_sc=False),
    )


def _tc_stage(inputs, body_fn, out_widths, rows=NPAD, bm=1024):
    grid = (rows // bm,)
    in_specs = []
    for a in inputs:
        if a.shape[0] == rows:
            in_specs.append(pl.BlockSpec((bm, a.shape[1]), lambda i: (i, 0)))
        else:
            in_specs.append(pl.BlockSpec(a.shape, lambda i: (0, 0)))
    out_shape = tuple(jax.ShapeDtypeStruct((rows, w), jnp.float32)
                      for w in out_widths)
    out_specs = tuple(pl.BlockSpec((bm, w), lambda i: (i, 0))
                      for w in out_widths)

    def kern(*refs):
        ins = refs[:len(inputs)]
        outs = refs[len(inputs):]
        vals = body_fn(*[r[...] for r in ins])
        if not isinstance(vals, tuple):
            vals = (vals,)
        for o, v in zip(outs, vals):
            o[...] = v

    return pl.pallas_call(
        kern, grid=grid, in_specs=in_specs, out_specs=out_specs,
        out_shape=out_shape)(*inputs)


def _dot(x, w):
    return jnp.dot(x, w, preferred_element_type=jnp.float32)


def _cat(a, b):
    return jnp.concatenate([a, b], axis=-1)


def _halves(x):
    h = x.shape[-1] // 2
    return x[:, :h], x[:, h:]


def kernel(x_field, mesh_x, boundary, edge_attr, edge_index, params):
    p = params

    def padN(a):
        return jnp.pad(a, ((0, NPAD - N), (0, 0)))

    xf = padN(x_field)
    mx = padN(mesh_x)
    bd = padN(boundary)
    ea = jnp.pad(edge_attr, ((0, EPAD - E), (0, 0)))
    srcp = jnp.pad(edge_index[0], (0, EPAD - E))
    dstp = jnp.pad(edge_index[1], (0, EPAD - E), constant_values=NPAD - 1)
    src16 = srcp.reshape(16, NS2, SUB)
    dst16 = dstp.reshape(16, NS2, SUB)
    src32 = srcp.reshape(2, 16, NS, SUB)
    dst32 = dstp.reshape(2, 16, NS, SUB)

    def agg(tl, tr):
        return _make_agg_split(tl.shape[1])(tl, tr, src16, dst16)

    def agg16(table):
        return _make_agg16()(table, src32, dst32)

    def b2(name):
        return p[name].reshape(1, -1)

    def bpad(name, d):
        b = p[name]
        return jnp.pad(b, (0, d - b.shape[0])).reshape(1, -1)

    def wpad(name, d):
        w = p[name]
        return jnp.pad(w, ((0, 0), (0, d - w.shape[1])))

    r = jax.nn.relu

    w_mesh_n = p["W_mesh"][:NM]
    w_mesh_e16 = jnp.pad(p["W_mesh"][NM:], ((0, 16 - NE), (0, 0)))
    ea16 = jnp.pad(ea, ((0, 0), (0, 16 - NE)))
    (yml, ymr) = _tc_stage(
        [mx, w_mesh_n],
        lambda x, w: _halves(_dot(x, w)),
        (64, 64))
    am = agg(yml, ymr)
    sa = _make_agg_lin16()(ea16, dst32)

    (m, y0, y1) = _tc_stage(
        [am[0], am[1], sa[0], sa[1], w_mesh_e16, b2("b_mesh"), p["W_u1"]],
        lambda a0, a1, s0, s1, we, b, w: (
            lambda mm: (mm,) + _halves(_dot(mm, w)))(
                r(_cat(a0, a1) + _dot(s0 + s1, we) + b)),
        (NM, 64, 64))

    a = agg(y0, y1)
    (u1, y0, y1) = _tc_stage(
        [a[0], a[1], b2("b_u1"), m, p["W_u2"]],
        lambda a0, a1, b, res, w: (
            lambda u: (u,) + _halves(_dot(u, w)))(r(_cat(a0, a1) + b) + res),
        (NM, 64, 64))
    a = agg(y0, y1)
    (u2, y0, y1) = _tc_stage(
        [a[0], a[1], b2("b_u2"), u1, p["W_u3"]],
        lambda a0, a1, b, res, w: (
            lambda u: (u,) + _halves(_dot(u, w)))(r(_cat(a0, a1) + b) + res),
        (NM, 64, 64))
    a = agg(y0, y1)
    wd10 = p["W_d10"]
    (y0, y1) = _tc_stage(
        [a[0], a[1], b2("b_u3"), u2, xf, bd, wd10[:NF], wd10[NF:NF + NB],
         wd10[NF + NB:]],
        lambda a0, a1, b, res, x, bdv, w1, w2, w3: _halves(
            (lambda u: _dot(x, w1) + _dot(bdv, w2) + _dot(u, w3))(
                r(_cat(a0, a1) + b) + res)),
        (32, 32))

    a = agg(y0, y1)
    (d0, y0, y1) = _tc_stage(
        [a[0], a[1], b2("b_d10"), p["W_d11"]],
        lambda a0, a1, b, w: (
            lambda x: (x,) + _halves(_dot(x, w)))(r(_cat(a0, a1) + b)),
        (64, 32, 32))
    a = agg(y0, y1)
    (y0, y1) = _tc_stage(
        [a[0], a[1], b2("b_d11"), p["W_d12"]],
        lambda a0, a1, b, w: _halves(_dot(r(_cat(a0, a1) + b), w)),
        (32, 32))
    a = agg(y0, y1)
    (y0, y1) = _tc_stage(
        [a[0], a[1], b2("b_d12"), d0],
        lambda a0, a1, b, res: _halves(r(_cat(a0, a1) + b) + res),
        (32, 32))

    a = agg(y0, y1)
    (e0, y0, y1) = _tc_stage(
        [a[0], a[1], p["W_d20"], b2("b_d20"), p["W_d21"]],
        lambda a0, a1, w0, b, w: (
            lambda x: (x,) + _halves(_dot(x, w)))(
                r(_dot(_cat(a0, a1), w0) + b)),
        (NM, 64, 64))
    a = agg(y0, y1)
    (y0, y1) = _tc_stage(
        [a[0], a[1], b2("b_d21"), p["W_d22"]],
        lambda a0, a1, b, w: _halves(_dot(r(_cat(a0, a1) + b), w)),
        (64, 64))
    a = agg(y0, y1)
    (y0, y1) = _tc_stage(
        [a[0], a[1], b2("b_d22"), e0, p["W_d30"]],
        lambda a0, a1, b, res, w: _halves(
            _dot(r(_cat(a0, a1) + b) + res, w)),
        (64, 64))

    a = agg(y0, y1)
    (y0, y1) = _tc_stage(
        [a[0], a[1], b2("b_d30"), p["W_d31"]],
        lambda a0, a1, b, w: _halves(_dot(r(_cat(a0, a1) + b), w)),
        (32, 32))
    a = agg(y0, y1)
    (y0, y1) = _tc_stage(
        [a[0], a[1], b2("b_d31"), p["W_d32"]],
        lambda a0, a1, b, w: _halves(_dot(r(_cat(a0, a1) + b), w)),
        (16, 16))
    a = agg(y0, y1)
    (yfd,) = _tc_stage(
        [a[0], a[1], b2("b_d32"), wpad("W_fdot", 16)],
        lambda a0, a1, b, w: _dot(r(_cat(a0, a1) + b), w),
        (16,))
    a = agg16(yfd)
    (fdot,) = _tc_stage(
        [a[0], a[1], bpad("b_fdot", 16)],
        lambda a0, a1, b: a0 + a1 + b,
        (16,))

    a = agg16(fdot)
    (i0, y0, y1) = _tc_stage(
        [a[0], a[1], jnp.pad(p["W_i10"], ((0, 8), (0, 0))), b2("b_i10"),
         p["W_i11"]],
        lambda a0, a1, w0, b, w: (
            lambda x: (x,) + _halves(_dot(x, w)))(r(_dot(a0 + a1, w0) + b)),
        (64, 32, 32))
    a = agg(y0, y1)
    (y0, y1) = _tc_stage(
        [a[0], a[1], b2("b_i11"), p["W_i12"]],
        lambda a0, a1, b, w: _halves(_dot(r(_cat(a0, a1) + b), w)),
        (32, 32))
    a = agg(y0, y1)
    (yio,) = _tc_stage(
        [a[0], a[1], b2("b_i12"), i0, wpad("W_iout", 16)],
        lambda a0, a1, b, res, w: _dot(r(_cat(a0, a1) + b) + res, w),
        (16,))
    a = agg16(yio)
    (out,) = _tc_stage(
        [a[0], a[1], bpad("b_iout", 16), xf],
        lambda a0, a1, b, x: x + (a0 + a1 + b)[:, :NF],
        (NF,))

    return out[:N]


# speedup vs baseline: 4.2958x; 1.0185x over previous
import functools

import jax
import jax.numpy as jnp
from jax import lax
from jax.experimental import pallas as pl
from jax.experimental.pallas import tpu as pltpu
from jax.experimental.pallas import tpu_sc as plsc

N = 10000
E = 160000
NF = 8
NB = 4
NM = 128
NE = 4

NPAD = 10240
SUB = 128
EPT = 5120
EPAD = 32 * EPT
EPT2 = 2 * EPT
NS = EPT // SUB
NS2 = EPT2 // SUB
ZR = NPAD // 16
ZB = 64


def _zero_acc(zblk, acc, s, dh):
    zf = jnp.zeros((16,), jnp.float32)

    def zb(i, carry):
        for kk in range(dh // 16):
            zblk[i, pl.ds(kk * 16, 16)] = zf
        return carry

    lax.fori_loop(0, ZB, zb, 0)
    for t in range(ZR // ZB):
        pltpu.sync_copy(zblk, acc.at[pl.ds(s * ZR + t * ZB, ZB)])


@functools.lru_cache(maxsize=None)
def _make_agg_split(dh):
    mesh = plsc.VectorSubcoreMesh(core_axis_name="c", subcore_axis_name="s")
    scratch = [
        pltpu.VMEM((NS2, SUB), jnp.int32),
        pltpu.VMEM((NS2, SUB), jnp.int32),
        pltpu.VMEM((SUB, dh), jnp.float32),
        pltpu.VMEM((ZB, dh), jnp.float32),
        pltpu.VMEM_SHARED((NPAD, dh), jnp.float32),
        pltpu.VMEM_SHARED((NPAD, dh), jnp.float32),
    ]

    def body(tl, tr, srcw, dstw, *rest):
        out, idx_s, idx_d, rows, zblk, tbl, acc = rest
        c = lax.axis_index("c")
        s = lax.axis_index("s")
        pltpu.sync_copy(srcw.at[s], idx_s)
        pltpu.sync_copy(dstw.at[s], idx_d)
        sl = pl.ds(s * ZR, ZR)

        @pl.when(c == 0)
        def _():
            pltpu.sync_copy(tl.at[sl], tbl.at[sl])

        @pl.when(c == 1)
        def _():
            pltpu.sync_copy(tr.at[sl], tbl.at[sl])

        _zero_acc(zblk, acc, s, dh)
        plsc.subcore_barrier()

        def step(j, carry):
            pltpu.sync_copy(tbl.at[idx_s.at[j]], rows)
            pltpu.sync_copy(rows, acc.at[idx_d.at[j]], add=True)
            return carry

        lax.fori_loop(0, NS2, step, 0)
        plsc.subcore_barrier()
        pltpu.sync_copy(acc.at[sl], out.at[c, sl])

    return pl.kernel(
        body,
        out_type=jax.ShapeDtypeStruct((2, NPAD, dh), jnp.float32),
        mesh=mesh,
        scratch_types=scratch,
        compiler_params=pltpu.CompilerParams(use_tc_tiling_on_sc=False),
    )


@functools.lru_cache(maxsize=None)
def _make_agg_lin16():
    d = 16
    mesh = plsc.VectorSubcoreMesh(core_axis_name="c", subcore_axis_name="s")
    scratch = [
        pltpu.VMEM((NS, SUB), jnp.int32),
        pltpu.VMEM((SUB, d), jnp.float32),
        pltpu.VMEM((ZB, d), jnp.float32),
        pltpu.VMEM_SHARED((NPAD, d), jnp.float32),
    ]

    def body(etab, dstw, out, idx_d, rows, zblk, acc):
        c = lax.axis_index("c")
        s = lax.axis_index("s")
        pltpu.sync_copy(dstw.at[c, s], idx_d)
        _zero_acc(zblk, acc, s, d)
        base = (c * 16 + s) * EPT
        plsc.subcore_barrier()

        def step(j, carry):
            pltpu.sync_copy(etab.at[pl.ds(base + j * SUB, SUB)], rows)
            pltpu.sync_copy(rows, acc.at[idx_d.at[j]], add=True)
            return carry

        lax.fori_loop(0, NS, step, 0)
        plsc.subcore_barrier()
        sl = pl.ds(s * ZR, ZR)
        pltpu.sync_copy(acc.at[sl], out.at[c, sl])

    return pl.kernel(
        body,
        out_type=jax.ShapeDtypeStruct((2, NPAD, d), jnp.float32),
        mesh=mesh,
        scratch_types=scratch,
        compiler_params=pltpu.CompilerParams(use_tc_tiling_on_sc=False),
    )


@functools.lru_cache(maxsize=None)
def _make_agg16sq():
    d = 16
    mesh = plsc.VectorSubcoreMesh(core_axis_name="c", subcore_axis_name="s")
    scratch = [
        pltpu.VMEM((NS2, SUB), jnp.int32),
        pltpu.VMEM((NS2, SUB), jnp.int32),
        pltpu.VMEM((SUB, d), jnp.float32),
        pltpu.VMEM((ZB, d), jnp.float32),
        pltpu.VMEM((ZR, d), jnp.float32),
        pltpu.VMEM_SHARED((NPAD, d), jnp.float32),
        pltpu.VMEM_SHARED((NPAD, d), jnp.float32),
    ]

    def body(table, srcw, dstw, out, idx_s, idx_d, rows, zblk, buf, tbl,
             acc):
        c = lax.axis_index("c")
        s = lax.axis_index("s")
        pltpu.sync_copy(srcw.at[s], idx_s)
        pltpu.sync_copy(dstw.at[s], idx_d)
        sl = pl.ds(s * ZR, ZR)
        pltpu.sync_copy(table.at[sl], tbl.at[sl])
        _zero_acc(zblk, acc, s, d)
        plsc.subcore_barrier()

        def step(j, carry):
            pltpu.sync_copy(tbl.at[idx_s.at[j]], rows)
            pltpu.sync_copy(rows, acc.at[idx_d.at[j]], add=True)
            return carry

        lax.fori_loop(0, NS2, step, 0)
        plsc.subcore_barrier()
        pltpu.sync_copy(acc.at[sl], buf)
        pltpu.sync_copy(buf, tbl.at[sl])
        _zero_acc(zblk, acc, s, d)
        plsc.subcore_barrier()
        lax.fori_loop(0, NS2, step, 0)
        plsc.subcore_barrier()

        @pl.when(c == 0)
        def _():
            pltpu.sync_copy(acc.at[sl], out.at[sl])

    return pl.kernel(
        body,
        out_type=jax.ShapeDtypeStruct((NPAD, d), jnp.float32),
        mesh=mesh,
        scratch_types=scratch,
        compiler_params=pltpu.CompilerParams(use_tc_tiling_on_sc=False),
    )


@functools.lru_cache(maxsize=None)
def _make_agg16():
    d = 16
    mesh = plsc.VectorSubcoreMesh(core_axis_name="c", subcore_axis_name="s")
    scratch = [
        pltpu.VMEM((NS, SUB), jnp.int32),
        pltpu.VMEM((NS, SUB), jnp.int32),
        pltpu.VMEM((SUB, d), jnp.float32),
        pltpu.VMEM((ZB, d), jnp.float32),
        pltpu.VMEM_SHARED((NPAD, d), jnp.float32),
        pltpu.VMEM_SHARED((NPAD, d), jnp.float32),
    ]

    def body(table, srcw, dstw, out, idx_s, idx_d, rows, zblk, tbl, acc):
        c = lax.axis_index("c")
        s = lax.axis_index("s")
        pltpu.sync_copy(srcw.at[c, s], idx_s)
        pltpu.sync_copy(dstw.at[c, s], idx_d)
        sl = pl.ds(s * ZR, ZR)
        pltpu.sync_copy(table.at[sl], tbl.at[sl])
        _zero_acc(zblk, acc, s, d)
        plsc.subcore_barrier()

        def step(j, carry):
            pltpu.sync_copy(tbl.at[idx_s.at[j]], rows)
            pltpu.sync_copy(rows, acc.at[idx_d.at[j]], add=True)
            return carry

        lax.fori_loop(0, NS, step, 0)
        plsc.subcore_barrier()
        pltpu.sync_copy(acc.at[sl], out.at[c, sl])

    return pl.kernel(
        body,
        out_type=jax.ShapeDtypeStruct((2, NPAD, d), jnp.float32),
        mesh=mesh,
        scratch_types=scratch,
        compiler_params=pltpu.CompilerParams(use_tc_tiling_on_sc=False),
    )


def _tc_stage(inputs, body_fn, out_widths, rows=NPAD, bm=1024):
    grid = (rows // bm,)
    in_specs = []
    for a in inputs:
        if a.shape[0] == rows:
            in_specs.append(pl.BlockSpec((bm, a.shape[1]), lambda i: (i, 0)))
        else:
            in_specs.append(pl.BlockSpec(a.shape, lambda i: (0, 0)))
    out_shape = tuple(jax.ShapeDtypeStruct((rows, w), jnp.float32)
                      for w in out_widths)
    out_specs = tuple(pl.BlockSpec((bm, w), lambda i: (i, 0))
                      for w in out_widths)

    def kern(*refs):
        ins = refs[:len(inputs)]
        outs = refs[len(inputs):]
        vals = body_fn(*[r[...] for r in ins])
        if not isinstance(vals, tuple):
            vals = (vals,)
        for o, v in zip(outs, vals):
            o[...] = v

    return pl.pallas_call(
        kern, grid=grid, in_specs=in_specs, out_specs=out_specs,
        out_shape=out_shape)(*inputs)


def _dot(x, w):
    return jnp.dot(x, w, preferred_element_type=jnp.float32)


def _cat(a, b):
    return jnp.concatenate([a, b], axis=-1)


def _halves(x):
    h = x.shape[-1] // 2
    return x[:, :h], x[:, h:]


def kernel(x_field, mesh_x, boundary, edge_attr, edge_index, params):
    p = params

    def padN(a):
        return jnp.pad(a, ((0, NPAD - N), (0, 0)))

    xf = padN(x_field)
    mx = padN(mesh_x)
    bd = padN(boundary)
    ea = jnp.pad(edge_attr, ((0, EPAD - E), (0, 0)))
    srcp = jnp.pad(edge_index[0], (0, EPAD - E))
    dstp = jnp.pad(edge_index[1], (0, EPAD - E), constant_values=NPAD - 1)
    src16 = srcp.reshape(16, NS2, SUB)
    dst16 = dstp.reshape(16, NS2, SUB)
    src32 = srcp.reshape(2, 16, NS, SUB)
    dst32 = dstp.reshape(2, 16, NS, SUB)

    def agg(tl, tr):
        return _make_agg_split(tl.shape[1])(tl, tr, src16, dst16)

    def agg16(table):
        return _make_agg16()(table, src32, dst32)

    def b2(name):
        return p[name].reshape(1, -1)

    def bpad(name, d):
        b = p[name]
        return jnp.pad(b, (0, d - b.shape[0])).reshape(1, -1)

    def wpad(name, d):
        w = p[name]
        return jnp.pad(w, ((0, 0), (0, d - w.shape[1])))

    r = jax.nn.relu

    w_mesh_n = p["W_mesh"][:NM]
    w_mesh_e16 = jnp.pad(p["W_mesh"][NM:], ((0, 16 - NE), (0, 0)))
    ea16 = jnp.concatenate(
        [ea, jnp.zeros((EPAD, 11), ea.dtype), jnp.ones((EPAD, 1), ea.dtype)],
        axis=1)
    (yml, ymr) = _tc_stage(
        [mx, w_mesh_n],
        lambda x, w: _halves(_dot(x, w)),
        (64, 64))
    am = agg(yml, ymr)
    sa = _make_agg_lin16()(ea16, dst32)

    (m, y0, y1) = _tc_stage(
        [am[0], am[1], sa[0], sa[1], w_mesh_e16, b2("b_mesh"), p["W_u1"]],
        lambda a0, a1, s0, s1, we, b, w: (
            lambda mm: (mm,) + _halves(_dot(mm, w)))(
                r(_cat(a0, a1) + _dot(s0 + s1, we) + b)),
        (NM, 64, 64))

    a = agg(y0, y1)
    (u1, y0, y1) = _tc_stage(
        [a[0], a[1], b2("b_u1"), m, p["W_u2"]],
        lambda a0, a1, b, res, w: (
            lambda u: (u,) + _halves(_dot(u, w)))(r(_cat(a0, a1) + b) + res),
        (NM, 64, 64))
    a = agg(y0, y1)
    (u2, y0, y1) = _tc_stage(
        [a[0], a[1], b2("b_u2"), u1, p["W_u3"]],
        lambda a0, a1, b, res, w: (
            lambda u: (u,) + _halves(_dot(u, w)))(r(_cat(a0, a1) + b) + res),
        (NM, 64, 64))
    a = agg(y0, y1)
    wd10 = p["W_d10"]
    (y0, y1) = _tc_stage(
        [a[0], a[1], b2("b_u3"), u2, xf, bd, wd10[:NF], wd10[NF:NF + NB],
         wd10[NF + NB:]],
        lambda a0, a1, b, res, x, bdv, w1, w2, w3: _halves(
            (lambda u: _dot(x, w1) + _dot(bdv, w2) + _dot(u, w3))(
                r(_cat(a0, a1) + b) + res)),
        (32, 32))

    a = agg(y0, y1)
    (d0, y0, y1) = _tc_stage(
        [a[0], a[1], b2("b_d10"), p["W_d11"]],
        lambda a0, a1, b, w: (
            lambda x: (x,) + _halves(_dot(x, w)))(r(_cat(a0, a1) + b)),
        (64, 32, 32))
    a = agg(y0, y1)
    (y0, y1) = _tc_stage(
        [a[0], a[1], b2("b_d11"), p["W_d12"]],
        lambda a0, a1, b, w: _halves(_dot(r(_cat(a0, a1) + b), w)),
        (32, 32))
    a = agg(y0, y1)
    (y0, y1) = _tc_stage(
        [a[0], a[1], b2("b_d12"), d0],
        lambda a0, a1, b, res: _halves(r(_cat(a0, a1) + b) + res),
        (32, 32))

    a = agg(y0, y1)
    (e0, y0, y1) = _tc_stage(
        [a[0], a[1], p["W_d20"], b2("b_d20"), p["W_d21"]],
        lambda a0, a1, w0, b, w: (
            lambda x: (x,) + _halves(_dot(x, w)))(
                r(_dot(_cat(a0, a1), w0) + b)),
        (NM, 64, 64))
    a = agg(y0, y1)
    (y0, y1) = _tc_stage(
        [a[0], a[1], b2("b_d21"), p["W_d22"]],
        lambda a0, a1, b, w: _halves(_dot(r(_cat(a0, a1) + b), w)),
        (64, 64))
    a = agg(y0, y1)
    (y0, y1) = _tc_stage(
        [a[0], a[1], b2("b_d22"), e0, p["W_d30"]],
        lambda a0, a1, b, res, w: _halves(
            _dot(r(_cat(a0, a1) + b) + res, w)),
        (64, 64))

    a = agg(y0, y1)
    (y0, y1) = _tc_stage(
        [a[0], a[1], b2("b_d30"), p["W_d31"]],
        lambda a0, a1, b, w: _halves(_dot(r(_cat(a0, a1) + b), w)),
        (32, 32))
    a = agg(y0, y1)
    (y0, y1) = _tc_stage(
        [a[0], a[1], b2("b_d31"), p["W_d32"]],
        lambda a0, a1, b, w: _halves(_dot(r(_cat(a0, a1) + b), w)),
        (16, 16))
    a = agg(y0, y1)
    (yfd,) = _tc_stage(
        [a[0], a[1], b2("b_d32"), wpad("W_fdot", 16)],
        lambda a0, a1, b, w: _dot(r(_cat(a0, a1) + b), w),
        (16,))

    aa = _make_agg16sq()(yfd, src16, dst16)
    indeg = sa[0][:, 15:16] + sa[1][:, 15:16]
    (i0, y0, y1) = _tc_stage(
        [aa, indeg, bpad("b_fdot", 16), jnp.pad(p["W_i10"], ((0, 8), (0, 0))),
         b2("b_i10"), p["W_i11"]],
        lambda x, dg, bf, w0, b, w: (
            lambda v: (v,) + _halves(_dot(v, w)))(
                r(_dot(x + dg * bf, w0) + b)),
        (64, 32, 32))
    a = agg(y0, y1)
    (y0, y1) = _tc_stage(
        [a[0], a[1], b2("b_i11"), p["W_i12"]],
        lambda a0, a1, b, w: _halves(_dot(r(_cat(a0, a1) + b), w)),
        (32, 32))
    a = agg(y0, y1)
    (yio,) = _tc_stage(
        [a[0], a[1], b2("b_i12"), i0, wpad("W_iout", 16)],
        lambda a0, a1, b, res, w: _dot(r(_cat(a0, a1) + b) + res, w),
        (16,))
    a = agg16(yio)
    (out,) = _tc_stage(
        [a[0], a[1], bpad("b_iout", 16), xf],
        lambda a0, a1, b, x: x + (a0 + a1 + b)[:, :NF],
        (NF,))

    return out[:N]
